# baseline (speedup 1.0000x reference)
import functools

import jax
import jax.numpy as jnp
from jax import lax
from jax.experimental import pallas as pl
from jax.experimental.pallas import tpu as pltpu
from jax.experimental.pallas import tpu_sc as plsc

_D = 128
_CHUNK = 128
_NBUF = 4


def _embed(ids2d, table):
    n_rows = ids2d.shape[0] * ids2d.shape[1]
    info = plsc.get_sparse_core_info()
    nc = info.num_cores
    nw = nc * info.num_subcores
    rows_w = n_rows // nw
    nch = rows_w // _CHUNK
    assert nch % _NBUF == 0

    mesh = plsc.VectorSubcoreMesh(core_axis_name="c", subcore_axis_name="s")

    @functools.partial(
        pl.kernel,
        mesh=mesh,
        out_type=jax.ShapeDtypeStruct((n_rows, _D), jnp.float32),
        scratch_types=(
            [pltpu.VMEM((nch, _CHUNK), jnp.int32)]
            + [pltpu.VMEM((_CHUNK, _D), jnp.float32)] * _NBUF
            + [pltpu.SemaphoreType.DMA] * (2 * _NBUF)
        ),
    )
    def emb(ids_hbm, table_hbm, out_hbm, idx_v, *bufs_and_sems):
        bufs = bufs_and_sems[:_NBUF]
        gsems = bufs_and_sems[_NBUF:2 * _NBUF]
        ssems = bufs_and_sems[2 * _NBUF:]
        wid = lax.axis_index("s") * nc + lax.axis_index("c")
        row0 = wid * rows_w
        pltpu.sync_copy(ids_hbm.at[pl.ds(wid * nch, nch)], idx_v)

        def gather_start(g, b):
            pltpu.async_copy(table_hbm.at[idx_v.at[g]], bufs[b], gsems[b])

        def gather_wait(g, b):
            pltpu.make_async_copy(
                table_hbm.at[idx_v.at[g]], bufs[b], gsems[b]).wait()

        def store_start(g, b):
            pltpu.async_copy(
                bufs[b], out_hbm.at[pl.ds(row0 + g * _CHUNK, _CHUNK)], ssems[b])

        def store_wait(g, b):
            pltpu.make_async_copy(
                bufs[b], out_hbm.at[pl.ds(row0 + g * _CHUNK, _CHUNK)],
                ssems[b]).wait()

        for b in range(_NBUF):
            gather_start(b, b)

        def body(i, carry):
            g0 = _NBUF * i
            for b in range(_NBUF):
                gather_wait(g0 + b, b)
                store_start(g0 + b, b)
            for b in range(_NBUF):
                store_wait(g0 + b, b)
                gather_start(g0 + _NBUF + b, b)
            return carry

        lax.fori_loop(0, nch // _NBUF - 1, body, 0)

        g0 = nch - _NBUF
        for b in range(_NBUF):
            gather_wait(g0 + b, b)
            store_start(g0 + b, b)
        for b in range(_NBUF):
            store_wait(g0 + b, b)

    return emb(ids2d, table)


def kernel(input_ids, table):
    b, l = input_ids.shape
    n = b * l
    ids2d = input_ids.astype(jnp.int32).reshape(n // _CHUNK, _CHUNK)
    out = _embed(ids2d, table)
    return out.reshape(b, l, _D)


# speedup vs baseline: 1.0040x; 1.0040x over previous
import functools

import jax
import jax.numpy as jnp
from jax import lax
from jax.experimental import pallas as pl
from jax.experimental.pallas import tpu as pltpu
from jax.experimental.pallas import tpu_sc as plsc

_D = 128
_IDXW = 128
_GPB = 2
_ROWS = _IDXW * _GPB


def _embed(ids2d, table):
    n_rows = ids2d.shape[0] * ids2d.shape[1]
    info = plsc.get_sparse_core_info()
    nc = info.num_cores
    nw = nc * info.num_subcores
    rows_w = n_rows // nw
    ng = rows_w // _IDXW
    nch = rows_w // _ROWS
    assert nch % 2 == 0

    mesh = plsc.VectorSubcoreMesh(core_axis_name="c", subcore_axis_name="s")

    @functools.partial(
        pl.kernel,
        mesh=mesh,
        out_type=jax.ShapeDtypeStruct((n_rows, _D), jnp.float32),
        scratch_types=[
            pltpu.VMEM((ng, _IDXW), jnp.int32),
            pltpu.VMEM((_ROWS, _D), jnp.float32),
            pltpu.VMEM((_ROWS, _D), jnp.float32),
            pltpu.SemaphoreType.DMA,
            pltpu.SemaphoreType.DMA,
            pltpu.SemaphoreType.DMA,
            pltpu.SemaphoreType.DMA,
        ],
    )
    def emb(ids_hbm, table_hbm, out_hbm, idx_v, buf_a, buf_b, ga, gb, sa, sb):
        wid = lax.axis_index("s") * nc + lax.axis_index("c")
        row0 = wid * rows_w
        pltpu.sync_copy(ids_hbm.at[pl.ds(wid * ng, ng)], idx_v)

        def gather_start(ch, buf, sem):
            for j in range(_GPB):
                pltpu.async_copy(
                    table_hbm.at[idx_v.at[_GPB * ch + j]],
                    buf.at[pl.ds(j * _IDXW, _IDXW)], sem)

        def gather_wait(ch, buf, sem):
            for j in range(_GPB):
                pltpu.make_async_copy(
                    table_hbm.at[idx_v.at[_GPB * ch + j]],
                    buf.at[pl.ds(j * _IDXW, _IDXW)], sem).wait()

        def store_start(ch, buf, sem):
            pltpu.async_copy(
                buf, out_hbm.at[pl.ds(row0 + ch * _ROWS, _ROWS)], sem)

        def store_wait(ch, buf, sem):
            pltpu.make_async_copy(
                buf, out_hbm.at[pl.ds(row0 + ch * _ROWS, _ROWS)], sem).wait()

        gather_start(0, buf_a, ga)
        gather_start(1, buf_b, gb)

        def body(i, carry):
            ch = 2 * i
            gather_wait(ch, buf_a, ga)
            store_start(ch, buf_a, sa)
            store_wait(ch, buf_a, sa)
            gather_start(ch + 2, buf_a, ga)
            gather_wait(ch + 1, buf_b, gb)
            store_start(ch + 1, buf_b, sb)
            store_wait(ch + 1, buf_b, sb)
            gather_start(ch + 3, buf_b, gb)
            return carry

        lax.fori_loop(0, nch // 2 - 1, body, 0)

        ch = nch - 2
        gather_wait(ch, buf_a, ga)
        store_start(ch, buf_a, sa)
        gather_wait(ch + 1, buf_b, gb)
        store_start(ch + 1, buf_b, sb)
        store_wait(ch, buf_a, sa)
        store_wait(ch + 1, buf_b, sb)

    return emb(ids2d, table)


def kernel(input_ids, table):
    b, l = input_ids.shape
    n = b * l
    ids2d = input_ids.astype(jnp.int32).reshape(n // _IDXW, _IDXW)
    out = _embed(ids2d, table)
    return out.reshape(b, l, _D)


# speedup vs baseline: 1.0064x; 1.0024x over previous
import functools

import jax
import jax.numpy as jnp
from jax import lax
from jax.experimental import pallas as pl
from jax.experimental.pallas import tpu as pltpu
from jax.experimental.pallas import tpu_sc as plsc

_D = 128
_CHUNK = 128
_NBUF = 4


def _embed(ids2d, table):
    n_rows = ids2d.shape[0] * ids2d.shape[1]
    info = plsc.get_sparse_core_info()
    nc = info.num_cores
    nw = nc * info.num_subcores
    rows_w = n_rows // nw
    nch = rows_w // _CHUNK
    assert nch % _NBUF == 0 and nch >= 2 * _NBUF

    mesh = plsc.VectorSubcoreMesh(core_axis_name="c", subcore_axis_name="s")

    @functools.partial(
        pl.kernel,
        mesh=mesh,
        out_type=jax.ShapeDtypeStruct((n_rows, _D), jnp.float32),
        scratch_types=(
            [pltpu.VMEM((nch, _CHUNK), jnp.int32)]
            + [pltpu.VMEM((_CHUNK, _D), jnp.float32)] * _NBUF
            + [pltpu.SemaphoreType.DMA] * (2 * _NBUF)
        ),
    )
    def emb(ids_hbm, table_hbm, out_hbm, idx_v, *bufs_and_sems):
        bufs = bufs_and_sems[:_NBUF]
        gsems = bufs_and_sems[_NBUF:2 * _NBUF]
        ssems = bufs_and_sems[2 * _NBUF:]
        wid = lax.axis_index("s") * nc + lax.axis_index("c")
        row0 = wid * rows_w
        pltpu.sync_copy(ids_hbm.at[pl.ds(wid * nch, nch)], idx_v)

        def gather_start(g, b):
            pltpu.async_copy(table_hbm.at[idx_v.at[g]], bufs[b], gsems[b])

        def gather_wait(g, b):
            pltpu.make_async_copy(
                table_hbm.at[idx_v.at[g]], bufs[b], gsems[b]).wait()

        def store_start(g, b):
            pltpu.async_copy(
                bufs[b], out_hbm.at[pl.ds(row0 + g * _CHUNK, _CHUNK)], ssems[b])

        def store_wait(g, b):
            pltpu.make_async_copy(
                bufs[b], out_hbm.at[pl.ds(row0 + g * _CHUNK, _CHUNK)],
                ssems[b]).wait()

        gather_start(0, 0)
        gather_start(1, 1)
        gather_wait(0, 0); store_start(0, 0); gather_start(2, 2)
        gather_wait(1, 1); store_start(1, 1); gather_start(3, 3)
        gather_wait(2, 2); store_start(2, 2); store_wait(0, 0); gather_start(4, 0)
        gather_wait(3, 3); store_start(3, 3); store_wait(1, 1); gather_start(5, 1)

        def body(i, carry):
            c0 = _NBUF * i
            for b in range(_NBUF):
                c = c0 + b
                gather_wait(c, b)
                store_start(c, b)
                store_wait(c - 2, (b - 2) % _NBUF)
                gather_start(c + 2, (b + 2) % _NBUF)
            return carry

        lax.fori_loop(1, nch // _NBUF - 1, body, 0)

        c = nch - 4
        gather_wait(c, 0); store_start(c, 0); store_wait(c - 2, 2); gather_start(c + 2, 2)
        gather_wait(c + 1, 1); store_start(c + 1, 1); store_wait(c - 1, 3); gather_start(c + 3, 3)
        gather_wait(c + 2, 2); store_start(c + 2, 2); store_wait(c, 0)
        gather_wait(c + 3, 3); store_start(c + 3, 3); store_wait(c + 1, 1)
        store_wait(c + 2, 2)
        store_wait(c + 3, 3)

    return emb(ids2d, table)


def kernel(input_ids, table):
    b, l = input_ids.shape
    n = b * l
    ids2d = input_ids.astype(jnp.int32).reshape(n // _CHUNK, _CHUNK)
    out = _embed(ids2d, table)
    return out.reshape(b, l, _D)
